# Initial kernel scaffold; baseline (speedup 1.0000x reference)
import jax
import jax.numpy as jnp
from jax.experimental import pallas as pl


def kernel(x, conv_w, conv_b, lin_w, lin_b):
    raise NotImplementedError("write your pallas kernel here")


# speedup vs baseline: 3.3553x; 3.3553x over previous
import jax
import jax.numpy as jnp
from jax.experimental import pallas as pl
from jax.experimental.pallas import tpu as pltpu


def _make_body(b_tile, Kt, c_in, c_out, T_out, V):
    N = T_out * V

    def _body(x_ref, w_ref, b2_ref, wt_ref, bl_ref, o_ref):
        wt = wt_ref[...]
        b2 = b2_ref[...]
        bl = bl_ref[...]
        ys = []
        for b in range(b_tile):
            xb = x_ref[b].astype(jnp.bfloat16)
            y = None
            for kt in range(Kt):
                w_kt = w_ref[:, kt * c_in:(kt + 1) * c_in]
                d = jax.lax.dot_general(
                    w_kt, xb[:, kt * V: kt * V + N],
                    dimension_numbers=(((1,), (0,)), ((), ())),
                    preferred_element_type=jnp.float32)
                y = d if y is None else y + d
            ys.append((y + b2).astype(jnp.bfloat16))
        for t in range(T_out):
            yt = jnp.concatenate(
                [ys[b][:, t * V:(t + 1) * V] for b in range(b_tile)], axis=0)
            zt = jax.lax.dot_general(
                yt, wt,
                dimension_numbers=(((1,), (0,)), ((), ())),
                preferred_element_type=jnp.float32) + bl
            for b in range(b_tile):
                o_ref[b, :, t * V:(t + 1) * V] = zt[b * c_out:(b + 1) * c_out]
    return _body


def kernel(x, conv_w, conv_b, lin_w, lin_b):
    B, c_in, T, V = x.shape
    c_out, _, Kt, _ = conv_w.shape
    T_out = T - Kt + 1
    N = T_out * V

    x2 = x.reshape(B, c_in, T * V)
    w_mat = jnp.transpose(conv_w[:, :, :, 0], (0, 2, 1)).reshape(
        c_out, Kt * c_in).astype(jnp.bfloat16)
    b2 = conv_b.reshape(c_out, 1).astype(jnp.float32)
    wt = lin_w.T.astype(jnp.bfloat16)
    bl = lin_b.reshape(1, V).astype(jnp.float32)

    b_tile = 4
    while B % b_tile:
        b_tile //= 2
    grid = (B // b_tile,)

    out = pl.pallas_call(
        _make_body(b_tile, Kt, c_in, c_out, T_out, V),
        out_shape=jax.ShapeDtypeStruct((B, c_out, N), jnp.float32),
        grid=grid,
        in_specs=[
            pl.BlockSpec((b_tile, c_in, T * V), lambda g: (g, 0, 0)),
            pl.BlockSpec((c_out, Kt * c_in), lambda g: (0, 0)),
            pl.BlockSpec((c_out, 1), lambda g: (0, 0)),
            pl.BlockSpec((V, V), lambda g: (0, 0)),
            pl.BlockSpec((1, V), lambda g: (0, 0)),
        ],
        out_specs=pl.BlockSpec((b_tile, c_out, N), lambda g: (g, 0, 0)),
        compiler_params=pltpu.CompilerParams(
            dimension_semantics=("parallel",),
            vmem_limit_bytes=64 * 1024 * 1024),
    )(x2, w_mat, b2, wt, bl)

    return out.reshape(B, c_out, T_out, V)
